# Initial kernel scaffold; baseline (speedup 1.0000x reference)
import jax
import jax.numpy as jnp
from jax.experimental import pallas as pl


def kernel(x, edge_index, edge_type, comp0, bases0, root0, bias0, comp1, bases1, root1, bias1, comp2, bases2, root2, bias2):
    raise NotImplementedError("write your pallas kernel here")


# speedup vs baseline: 3.0096x; 3.0096x over previous
import functools

import jax
import jax.numpy as jnp
from jax import lax
from jax.experimental import pallas as pl
from jax.experimental.pallas import tpu as pltpu
from jax.experimental.pallas import tpu_sc as plsc

H = 128
ACCW = 64
R = 8
CHUNK = 128


def _gather_pass(n_tab, width, e_rows):
  mesh = plsc.VectorSubcoreMesh(core_axis_name="c", subcore_axis_name="s")
  rows_per_tile = e_rows // 32
  chunks_per_tile = rows_per_tile // CHUNK
  assert rows_per_tile % CHUNK == 0

  @functools.partial(
      pl.kernel,
      out_type=jax.ShapeDtypeStruct((e_rows, width), jnp.float32),
      mesh=mesh,
      scratch_types=[
          pltpu.VMEM((CHUNK,), jnp.int32),
          pltpu.VMEM((CHUNK, width), jnp.float32),
          pltpu.SemaphoreType.DMA,
      ],
  )
  def k(tab_hbm, src_hbm, msg_hbm, src_v, rows_v, sem):
    wid = lax.axis_index("c") * 16 + lax.axis_index("s")
    base = wid * rows_per_tile

    def chunk_body(g, _):
      eb = pl.multiple_of(base + g * CHUNK, CHUNK)
      pltpu.sync_copy(src_hbm.at[pl.ds(eb, CHUNK)], src_v)
      pltpu.async_copy(tab_hbm.at[src_v], rows_v, sem).wait()
      pltpu.sync_copy(rows_v, msg_hbm.at[pl.ds(eb, CHUNK)])
      return 0
    lax.fori_loop(0, chunks_per_tile, chunk_body, 0)

  return k


def _scatter_pass(n_half, n_init, e_rows):
  mesh = plsc.VectorSubcoreMesh(core_axis_name="c", subcore_axis_name="s")
  zrows = 64
  init_per_tile = n_init // 16
  out_per_tile = n_half // 16
  assert init_per_tile % zrows == 0 and out_per_tile % zrows == 0
  chunks_per_tile = e_rows // (16 * CHUNK)
  assert e_rows % (16 * CHUNK) == 0

  @functools.partial(
      pl.kernel,
      out_type=jax.ShapeDtypeStruct((2 * n_half, H), jnp.float32),
      mesh=mesh,
      scratch_types=[
          pltpu.VMEM((CHUNK,), jnp.int32),
          pltpu.VMEM((CHUNK, H), jnp.float32),
          pltpu.VMEM((zrows, H), jnp.float32),
          pltpu.VMEM_SHARED((n_init, H), jnp.float32),
      ],
  )
  def k(scaled_hbm, dstc_hbm, out_hbm, dst_v, chunk_v, zbuf_v, acc_sh):
    cid = lax.axis_index("c")
    sid = lax.axis_index("s")

    def zrow(i, _):
      for j in range(H // 16):
        zbuf_v[i, pl.ds(16 * j, 16)] = jnp.zeros((16,), jnp.float32)
      return 0
    lax.fori_loop(0, zrows, zrow, 0)
    for b in range(init_per_tile // zrows):
      pltpu.sync_copy(zbuf_v, acc_sh.at[pl.ds(sid * init_per_tile + b * zrows, zrows)])
    plsc.subcore_barrier()

    base = sid * (chunks_per_tile * CHUNK)

    def chunk_body(g, _):
      eb = pl.multiple_of(base + g * CHUNK, CHUNK)
      db = pl.multiple_of(cid * e_rows + eb, CHUNK)
      pltpu.sync_copy(dstc_hbm.at[pl.ds(db, CHUNK)], dst_v)
      pltpu.sync_copy(scaled_hbm.at[pl.ds(eb, CHUNK)], chunk_v)
      pltpu.sync_copy(chunk_v, acc_sh.at[dst_v], add=True)
      return 0
    lax.fori_loop(0, chunks_per_tile, chunk_body, 0)
    plsc.subcore_barrier()

    for b in range(out_per_tile // zrows):
      r = sid * out_per_tile + b * zrows
      pltpu.sync_copy(acc_sh.at[pl.ds(r, zrows)], zbuf_v)
      ro = pl.multiple_of(cid * n_half + r, zrows)
      pltpu.sync_copy(zbuf_v, out_hbm.at[pl.ds(ro, zrows)])

  return k


def _matmul(x, w):
  n, k = x.shape
  p = w.shape[1]
  bn = 512
  assert n % bn == 0

  def body(xr, wr, outr):
    outr[...] = jnp.dot(xr[...], wr[...], preferred_element_type=jnp.float32)

  return pl.pallas_call(
      body,
      grid=(n // bn,),
      in_specs=[
          pl.BlockSpec((bn, k), lambda i: (i, 0)),
          pl.BlockSpec((k, p), lambda i: (0, 0)),
      ],
      out_specs=pl.BlockSpec((bn, p), lambda i: (i, 0)),
      out_shape=jax.ShapeDtypeStruct((n, p), jnp.float32),
  )(x, w)


def kernel(x, edge_index, edge_type, comp0, bases0, root0, bias0,
           comp1, bases1, root1, bias1, comp2, bases2, root2, bias2):
  n, h = x.shape
  src = edge_index[0]
  dst = edge_index[1]
  et = edge_type
  e = src.shape[0]
  n_pad = ((n + 2047) // 2048) * 2048
  x = jnp.pad(x, ((0, n_pad - n), (0, 0)))

  flat = dst * R + et
  cnt = jnp.zeros((n * R,), jnp.float32).at[flat].add(1.0)
  inv = 1.0 / jnp.maximum(cnt, 1.0)
  ic = inv[flat]

  e_pad = ((e + 8191) // 8192) * 8192
  pad = e_pad - e
  src_p = jnp.pad(src, (0, pad))
  dst_p = jnp.pad(dst, (0, pad), constant_values=n)

  n_half = n_pad // 2
  n_init = n_half + 1024
  d0 = jnp.where(dst_p < n_half, dst_p, n_half)
  d1 = jnp.where(dst_p >= n_half, dst_p - n_half, n_half)
  d1 = jnp.where(dst_p >= n, n_half, d1)
  dst_cat = jnp.concatenate([d0, d1])

  def coefs(comp):
    cf = comp[et, :] * ic[:, None]
    cf = jnp.pad(cf, ((0, pad), (0, 0)))
    return cf[:, 0:1], cf[:, 1:2]

  gather_sq = _gather_pass(n_pad, 2 * H, e_pad)
  gather_l2 = _gather_pass(n_pad, H, e_pad)
  scatter = _scatter_pass(n_half, n_init, e_pad)

  def square_layer(xl, comp, bases, root, bias):
    wcat = jnp.concatenate([bases[0], bases[1], root], axis=1)
    y = _matmul(xl, wcat)
    msg = gather_sq(y[:, :2 * H], src_p)
    cf0, cf1 = coefs(comp)
    scaled = msg[:, :H] * cf0 + msg[:, H:] * cf1
    agg = scatter(scaled, dst_cat)
    return y[:, 2 * H:] + agg + bias

  x1 = square_layer(x, comp0, bases0, root0, bias0)
  x2 = square_layer(x1, comp1, bases1, root1, bias1)

  w2 = jnp.concatenate([bases2[0], bases2[1], root2], axis=1)
  w2p = jnp.pad(w2, ((0, 0), (0, H - 3)))
  y2 = _matmul(x2, w2p)
  msg2 = gather_l2(y2, src_p)
  cf20, cf21 = coefs(comp2)
  val = msg2[:, 0:1] * cf20 + msg2[:, 1:2] * cf21
  scaled2 = jnp.broadcast_to(val, (e_pad, H))
  acc2 = scatter(scaled2, dst_cat)
  return (y2[:n, 2:3] + acc2[:n, :1]) + bias2
